# baseline (speedup 1.0000x reference)
import functools

import jax
import jax.numpy as jnp
from jax import lax
from jax.experimental import pallas as pl
from jax.experimental.pallas import tpu as pltpu
from jax.experimental.pallas import tpu_sc as plsc

D = 32
B = 4096
NC, NS, L = 2, 16, 16
NW = NC * NS
BPW = B // NW
GROUPS = BPW // L

_sc_mesh = plsc.VectorSubcoreMesh(core_axis_name="c", subcore_axis_name="s")


N = 1000000


@functools.partial(
    pl.kernel,
    out_type=jax.ShapeDtypeStruct((B,), jnp.float32),
    mesh=_sc_mesh,
    compiler_params=pltpu.CompilerParams(use_tc_tiling_on_sc=False),
    scratch_types=[
        pltpu.VMEM((BPW,), jnp.int32),
        pltpu.VMEM((BPW,), jnp.int32),
        pltpu.VMEM((D, BPW), jnp.int32),
        pltpu.VMEM((D, BPW), jnp.int32),
        pltpu.VMEM((D, BPW), jnp.float32),
        pltpu.VMEM((D, BPW), jnp.float32),
        pltpu.VMEM((BPW,), jnp.float32),
        pltpu.SemaphoreType.DMA,
        pltpu.SemaphoreType.DMA,
    ],
)
def _sc_dot(uflat, iflat, uids, iids,
            dot_out,
            uid_v, iid_v, uidx, iidx, udT, idT, dotv,
            sem_u, sem_i):
    wid = lax.axis_index("s") * NC + lax.axis_index("c")
    base = wid * BPW
    pltpu.sync_copy(uids.at[pl.ds(base, BPW)], uid_v)
    pltpu.sync_copy(iids.at[pl.ds(base, BPW)], iid_v)
    for d in range(D):
        for g in range(GROUPS):
            s = pl.ds(g * L, L)
            uidx[d, s] = uid_v[s] + (d * N)
            iidx[d, s] = iid_v[s] + (d * N)
    cus = [pltpu.async_copy(uflat.at[uidx.at[d]], udT.at[d], sem_u)
           for d in range(D)]
    cis = [pltpu.async_copy(iflat.at[iidx.at[d]], idT.at[d], sem_i)
           for d in range(D)]
    for c in cus:
        c.wait()
    for c in cis:
        c.wait()
    for g in range(GROUPS):
        s = pl.ds(g * L, L)
        acc = udT[0, s] * idT[0, s]
        for d in range(1, D):
            acc = acc + udT[d, s] * idT[d, s]
        dotv[s] = acc
    pltpu.sync_copy(dotv, dot_out.at[pl.ds(base, BPW)])


@functools.partial(
    pl.kernel,
    out_type=jax.ShapeDtypeStruct((B,), jnp.float32),
    mesh=_sc_mesh,
    compiler_params=pltpu.CompilerParams(use_tc_tiling_on_sc=False),
    scratch_types=[
        pltpu.VMEM((BPW,), jnp.int32),
        pltpu.VMEM((BPW,), jnp.int32),
        pltpu.VMEM((BPW,), jnp.float32),
        pltpu.VMEM((BPW,), jnp.float32),
        pltpu.SemaphoreType.DMA,
        pltpu.SemaphoreType.DMA,
    ],
)
def _sc_bias(ubias, ibias, uids, iids, brow_out,
             uid_v, iid_v, ubv, ibv, sem_ub, sem_ib):
    wid = lax.axis_index("s") * NC + lax.axis_index("c")
    base = wid * BPW
    pltpu.sync_copy(uids.at[pl.ds(base, BPW)], uid_v)
    pltpu.sync_copy(iids.at[pl.ds(base, BPW)], iid_v)
    cub = pltpu.async_copy(ubias.at[uid_v], ubv, sem_ub)
    cib = pltpu.async_copy(ibias.at[iid_v], ibv, sem_ib)
    cub.wait()
    cib.wait()
    for g in range(GROUPS):
        s = pl.ds(g * L, L)
        ubv[s] = ubv[s] + ibv[s]
    pltpu.sync_copy(ubv, brow_out.at[pl.ds(base, BPW)])


def _bcast_body(dot_ref, brow_ref, out_ref):
    out_ref[...] = brow_ref[...] + dot_ref[...]


TILE_I = 512


@jax.jit
def _tc_stage(dot, brow):
    return pl.pallas_call(
        _bcast_body,
        grid=(B // TILE_I,),
        in_specs=[
            pl.BlockSpec((1, B), lambda i: (0, 0)),
            pl.BlockSpec((TILE_I, 1), lambda i: (i, 0)),
        ],
        out_specs=pl.BlockSpec((TILE_I, B), lambda i: (i, 0)),
        out_shape=jax.ShapeDtypeStruct((B, B), jnp.float32),
    )(dot.reshape(1, B), brow.reshape(B, 1))


def kernel(user_ids, item_ids, user_emb, item_emb, user_bias, item_bias):
    uids = user_ids.astype(jnp.int32)
    iids = item_ids.astype(jnp.int32)
    dot = _sc_dot(user_emb.T.reshape(-1), item_emb.T.reshape(-1), uids, iids)
    brow = _sc_bias(user_bias.reshape(-1), item_bias.reshape(-1), uids, iids)
    return _tc_stage(dot, brow)


# speedup vs baseline: 5.6091x; 5.6091x over previous
import functools

import jax
import jax.numpy as jnp
from jax import lax
from jax.experimental import pallas as pl
from jax.experimental.pallas import tpu as pltpu
from jax.experimental.pallas import tpu_sc as plsc

D = 32
B = 4096
NC, NS, L = 2, 16, 16
NW = NC * NS
BPW = B // NW
GROUPS = BPW // L

_sc_mesh = plsc.VectorSubcoreMesh(core_axis_name="c", subcore_axis_name="s")


@functools.partial(
    pl.kernel,
    out_type=(
        jax.ShapeDtypeStruct((B, D), jnp.float32),
        jax.ShapeDtypeStruct((B, D), jnp.float32),
        jax.ShapeDtypeStruct((B,), jnp.float32),
    ),
    mesh=_sc_mesh,
    compiler_params=pltpu.CompilerParams(use_tc_tiling_on_sc=False),
    scratch_types=[
        pltpu.VMEM((BPW,), jnp.int32),
        pltpu.VMEM((BPW,), jnp.int32),
        pltpu.VMEM((BPW, D), jnp.float32),
        pltpu.VMEM((BPW, D), jnp.float32),
        pltpu.VMEM((BPW,), jnp.float32),
        pltpu.VMEM((BPW,), jnp.float32),
        pltpu.SemaphoreType.DMA,
        pltpu.SemaphoreType.DMA,
        pltpu.SemaphoreType.DMA,
        pltpu.SemaphoreType.DMA,
    ],
)
def _sc_gather(uemb, iemb, uids, iids, ubias, ibias,
               ue_out, ie_out, brow_out,
               uid_v, iid_v, urows, irows, ubv, ibv,
               sem_u, sem_i, sem_ub, sem_ib):
    wid = lax.axis_index("s") * NC + lax.axis_index("c")
    base = wid * BPW
    pltpu.sync_copy(uids.at[pl.ds(base, BPW)], uid_v)
    pltpu.sync_copy(iids.at[pl.ds(base, BPW)], iid_v)
    cu = pltpu.async_copy(uemb.at[uid_v], urows, sem_u)
    ci = pltpu.async_copy(iemb.at[iid_v], irows, sem_i)
    cub = pltpu.async_copy(ubias.at[uid_v], ubv, sem_ub)
    cib = pltpu.async_copy(ibias.at[iid_v], ibv, sem_ib)
    cu.wait()
    pltpu.sync_copy(urows, ue_out.at[pl.ds(base, BPW)])
    ci.wait()
    pltpu.sync_copy(irows, ie_out.at[pl.ds(base, BPW)])
    cub.wait()
    cib.wait()
    for g in range(GROUPS):
        s = pl.ds(g * L, L)
        ubv[s] = ubv[s] + ibv[s]
    pltpu.sync_copy(ubv, brow_out.at[pl.ds(base, BPW)])


def _dot_body(ue_ref, ie_ref, o_ref):
    o_ref[...] = jnp.sum(ue_ref[...] * ie_ref[...], axis=1, keepdims=True)


def _bcast_body(dot_ref, brow_ref, out_ref):
    out_ref[...] = brow_ref[...] + dot_ref[...]


TILE_I = 512


@jax.jit
def _tc_stage(ue, ie, brow):
    dot_col = pl.pallas_call(
        _dot_body,
        out_shape=jax.ShapeDtypeStruct((B, 1), jnp.float32),
    )(ue, ie)
    return pl.pallas_call(
        _bcast_body,
        grid=(B // TILE_I,),
        in_specs=[
            pl.BlockSpec((1, B), lambda i: (0, 0)),
            pl.BlockSpec((TILE_I, 1), lambda i: (i, 0)),
        ],
        out_specs=pl.BlockSpec((TILE_I, B), lambda i: (i, 0)),
        out_shape=jax.ShapeDtypeStruct((B, B), jnp.float32),
    )(dot_col.reshape(1, B), brow.reshape(B, 1))


def kernel(user_ids, item_ids, user_emb, item_emb, user_bias, item_bias):
    uids = user_ids.astype(jnp.int32)
    iids = item_ids.astype(jnp.int32)
    ue, ie, brow = _sc_gather(
        user_emb, item_emb, uids, iids,
        user_bias.reshape(-1), item_bias.reshape(-1))
    return _tc_stage(ue, ie, brow)


# speedup vs baseline: 7.1037x; 1.2665x over previous
import functools

import jax
import jax.numpy as jnp
from jax import lax
from jax.experimental import pallas as pl
from jax.experimental.pallas import tpu as pltpu
from jax.experimental.pallas import tpu_sc as plsc

D = 32
B = 4096
N = 1000000
PACK = 4
WIDE = PACK * D
SUBL = 1024
NC, NS, L = 2, 16, 16
NW = NC * NS
BPW = B // NW
GROUPS = BPW // L

TRC = PACK * SUBL
TRG = -(-N // TRC)
NWR = TRG * SUBL

_sc_mesh = plsc.VectorSubcoreMesh(core_axis_name="c", subcore_axis_name="s")


def _tr_body(in_ref, o_ref):
    o_ref[...] = jnp.concatenate(
        [in_ref[:, q * SUBL:(q + 1) * SUBL].T for q in range(PACK)], axis=1)


@jax.jit
def _to_wide(embT):
    return pl.pallas_call(
        _tr_body,
        grid=(TRG,),
        in_specs=[pl.BlockSpec((D, TRC), lambda i: (0, i))],
        out_specs=pl.BlockSpec((SUBL, WIDE), lambda i: (i, 0)),
        out_shape=jax.ShapeDtypeStruct((NWR, WIDE), jnp.float32),
    )(embT)


@functools.partial(
    pl.kernel,
    out_type=(
        jax.ShapeDtypeStruct((B, WIDE), jnp.float32),
        jax.ShapeDtypeStruct((B, WIDE), jnp.float32),
        jax.ShapeDtypeStruct((B,), jnp.float32),
    ),
    mesh=_sc_mesh,
    scratch_types=[
        pltpu.VMEM((BPW,), jnp.int32),
        pltpu.VMEM((BPW,), jnp.int32),
        pltpu.VMEM((BPW,), jnp.int32),
        pltpu.VMEM((BPW,), jnp.int32),
        pltpu.VMEM((BPW, WIDE), jnp.float32),
        pltpu.VMEM((BPW, WIDE), jnp.float32),
        pltpu.VMEM((BPW,), jnp.float32),
        pltpu.VMEM((BPW,), jnp.float32),
        pltpu.SemaphoreType.DMA,
        pltpu.SemaphoreType.DMA,
        pltpu.SemaphoreType.DMA,
        pltpu.SemaphoreType.DMA,
    ],
)
def _sc_gather(uemb2, iemb2, uids, iids, ubias, ibias,
               ue_out, ie_out, brow_out,
               uid_v, iid_v, utid_v, itid_v, uwide, iwide, ubv, ibv,
               sem_u, sem_i, sem_ub, sem_ib):
    wid = lax.axis_index("s") * NC + lax.axis_index("c")
    base = wid * BPW
    pltpu.sync_copy(uids.at[pl.ds(base, BPW)], uid_v)
    pltpu.sync_copy(iids.at[pl.ds(base, BPW)], iid_v)
    for g in range(GROUPS):
        s = pl.ds(g * L, L)
        u = uid_v[s]
        i = iid_v[s]
        utid_v[s] = lax.shift_left(lax.shift_right_logical(u, 12), 10) + \
            (u & (SUBL - 1))
        itid_v[s] = lax.shift_left(lax.shift_right_logical(i, 12), 10) + \
            (i & (SUBL - 1))
    cu = pltpu.async_copy(uemb2.at[utid_v], uwide, sem_u)
    ci = pltpu.async_copy(iemb2.at[itid_v], iwide, sem_i)
    cub = pltpu.async_copy(ubias.at[uid_v], ubv, sem_ub)
    cib = pltpu.async_copy(ibias.at[iid_v], ibv, sem_ib)
    cu.wait()
    pltpu.sync_copy(uwide, ue_out.at[pl.ds(base, BPW)])
    ci.wait()
    pltpu.sync_copy(iwide, ie_out.at[pl.ds(base, BPW)])
    cub.wait()
    cib.wait()
    for g in range(GROUPS):
        s = pl.ds(g * L, L)
        ubv[s] = ubv[s] + ibv[s]
    pltpu.sync_copy(ubv, brow_out.at[pl.ds(base, BPW)])


def _dot_body(uw_ref, iw_ref, uq_ref, iq_ref, o_ref):
    uq = lax.shift_right_logical(uq_ref[...], 10) & 3
    iq = lax.shift_right_logical(iq_ref[...], 10) & 3
    ue = jnp.zeros((B, D), jnp.float32)
    ie = jnp.zeros((B, D), jnp.float32)
    for q in range(PACK):
        sel = pl.ds(q * D, D)
        ue = ue + jnp.where(uq == q, uw_ref[:, sel], 0.0)
        ie = ie + jnp.where(iq == q, iw_ref[:, sel], 0.0)
    o_ref[...] = jnp.sum(ue * ie, axis=1, keepdims=True)


def _bcast_body(dot_ref, brow_ref, out_ref):
    out_ref[...] = brow_ref[...] + dot_ref[...]


TILE_I = 512


@jax.jit
def _tc_stage(uw, iw, uids, iids, brow):
    dot_col = pl.pallas_call(
        _dot_body,
        out_shape=jax.ShapeDtypeStruct((B, 1), jnp.float32),
    )(uw, iw, uids.reshape(B, 1), iids.reshape(B, 1))
    return pl.pallas_call(
        _bcast_body,
        grid=(B // TILE_I,),
        in_specs=[
            pl.BlockSpec((1, B), lambda i: (0, 0)),
            pl.BlockSpec((TILE_I, 1), lambda i: (i, 0)),
        ],
        out_specs=pl.BlockSpec((TILE_I, B), lambda i: (i, 0)),
        out_shape=jax.ShapeDtypeStruct((B, B), jnp.float32),
    )(dot_col.reshape(1, B), brow.reshape(B, 1))


def kernel(user_ids, item_ids, user_emb, item_emb, user_bias, item_bias):
    uids = user_ids.astype(jnp.int32)
    iids = item_ids.astype(jnp.int32)
    uw2 = _to_wide(user_emb.T)
    iw2 = _to_wide(item_emb.T)
    uw, iw, brow = _sc_gather(
        uw2, iw2, uids, iids,
        user_bias.reshape(-1), item_bias.reshape(-1))
    return _tc_stage(uw, iw, uids, iids, brow)


# speedup vs baseline: 8.0775x; 1.1371x over previous
import functools

import jax
import jax.numpy as jnp
from jax import lax
from jax.experimental import pallas as pl
from jax.experimental.pallas import tpu as pltpu
from jax.experimental.pallas import tpu_sc as plsc

D = 32
B = 4096
N = 1000000
PACK = 4
WIDE = PACK * D
SUBL = 2048
SHS = 11
NC, NS, L = 2, 16, 16
NW = NC * NS
BPW = B // NW
GROUPS = BPW // L

TRC = PACK * SUBL
TRG = -(-N // TRC)
NWR = TRG * SUBL

_sc_mesh = plsc.VectorSubcoreMesh(core_axis_name="c", subcore_axis_name="s")


def _tr_body(in_ref, o_ref):
    for q in range(PACK):
        o_ref[:, q * D:(q + 1) * D] = in_ref[:, q * SUBL:(q + 1) * SUBL].T


@jax.jit
def _to_wide(embT):
    return pl.pallas_call(
        _tr_body,
        grid=(TRG,),
        in_specs=[pl.BlockSpec((D, TRC), lambda i: (0, i))],
        out_specs=pl.BlockSpec((SUBL, WIDE), lambda i: (i, 0)),
        out_shape=jax.ShapeDtypeStruct((NWR, WIDE), jnp.float32),
    )(embT)


@functools.partial(
    pl.kernel,
    out_type=(
        jax.ShapeDtypeStruct((B, WIDE), jnp.float32),
        jax.ShapeDtypeStruct((B, WIDE), jnp.float32),
        jax.ShapeDtypeStruct((B,), jnp.float32),
    ),
    mesh=_sc_mesh,
    scratch_types=[
        pltpu.VMEM((BPW,), jnp.int32),
        pltpu.VMEM((BPW,), jnp.int32),
        pltpu.VMEM((BPW,), jnp.int32),
        pltpu.VMEM((BPW,), jnp.int32),
        pltpu.VMEM((BPW, WIDE), jnp.float32),
        pltpu.VMEM((BPW, WIDE), jnp.float32),
        pltpu.VMEM((BPW,), jnp.float32),
        pltpu.VMEM((BPW,), jnp.float32),
        pltpu.SemaphoreType.DMA,
        pltpu.SemaphoreType.DMA,
        pltpu.SemaphoreType.DMA,
        pltpu.SemaphoreType.DMA,
    ],
)
def _sc_gather(uemb2, iemb2, uids, iids, ubias, ibias,
               ue_out, ie_out, brow_out,
               uid_v, iid_v, utid_v, itid_v, uwide, iwide, ubv, ibv,
               sem_u, sem_i, sem_ub, sem_ib):
    wid = lax.axis_index("s") * NC + lax.axis_index("c")
    base = wid * BPW
    pltpu.sync_copy(uids.at[pl.ds(base, BPW)], uid_v)
    pltpu.sync_copy(iids.at[pl.ds(base, BPW)], iid_v)
    for g in range(GROUPS):
        s = pl.ds(g * L, L)
        u = uid_v[s]
        i = iid_v[s]
        utid_v[s] = lax.shift_left(lax.shift_right_logical(u, SHS + 2), SHS) \
            + (u & (SUBL - 1))
        itid_v[s] = lax.shift_left(lax.shift_right_logical(i, SHS + 2), SHS) \
            + (i & (SUBL - 1))
    cu = pltpu.async_copy(uemb2.at[utid_v], uwide, sem_u)
    ci = pltpu.async_copy(iemb2.at[itid_v], iwide, sem_i)
    cub = pltpu.async_copy(ubias.at[uid_v], ubv, sem_ub)
    cib = pltpu.async_copy(ibias.at[iid_v], ibv, sem_ib)
    cu.wait()
    pltpu.sync_copy(uwide, ue_out.at[pl.ds(base, BPW)])
    ci.wait()
    pltpu.sync_copy(iwide, ie_out.at[pl.ds(base, BPW)])
    cub.wait()
    cib.wait()
    for g in range(GROUPS):
        s = pl.ds(g * L, L)
        ubv[s] = ubv[s] + ibv[s]
    pltpu.sync_copy(ubv, brow_out.at[pl.ds(base, BPW)])


def _dot_body(uw_ref, iw_ref, uq_ref, iq_ref, o_ref):
    uq = lax.shift_right_logical(uq_ref[...], SHS) & 3
    iq = lax.shift_right_logical(iq_ref[...], SHS) & 3
    ue = jnp.zeros((B, D), jnp.float32)
    ie = jnp.zeros((B, D), jnp.float32)
    for q in range(PACK):
        sel = pl.ds(q * D, D)
        ue = ue + jnp.where(uq == q, uw_ref[:, sel], 0.0)
        ie = ie + jnp.where(iq == q, iw_ref[:, sel], 0.0)
    o_ref[...] = jnp.sum(ue * ie, axis=1, keepdims=True)


def _bcast_body(dot_ref, brow_ref, out_ref):
    out_ref[...] = brow_ref[...] + dot_ref[...]


TILE_I = 512


@jax.jit
def _tc_stage(uw, iw, uids, iids, brow):
    dot_col = pl.pallas_call(
        _dot_body,
        out_shape=jax.ShapeDtypeStruct((B, 1), jnp.float32),
    )(uw, iw, uids.reshape(B, 1), iids.reshape(B, 1))
    return pl.pallas_call(
        _bcast_body,
        grid=(B // TILE_I,),
        in_specs=[
            pl.BlockSpec((1, B), lambda i: (0, 0)),
            pl.BlockSpec((TILE_I, 1), lambda i: (i, 0)),
        ],
        out_specs=pl.BlockSpec((TILE_I, B), lambda i: (i, 0)),
        out_shape=jax.ShapeDtypeStruct((B, B), jnp.float32),
    )(dot_col.reshape(1, B), brow.reshape(B, 1))


def kernel(user_ids, item_ids, user_emb, item_emb, user_bias, item_bias):
    uids = user_ids.astype(jnp.int32)
    iids = item_ids.astype(jnp.int32)
    uw2 = _to_wide(user_emb.T)
    iw2 = _to_wide(item_emb.T)
    uw, iw, brow = _sc_gather(
        uw2, iw2, uids, iids,
        user_bias.reshape(-1), item_bias.reshape(-1))
    return _tc_stage(uw, iw, uids, iids, brow)


# speedup vs baseline: 8.1696x; 1.0114x over previous
import functools

import jax
import jax.numpy as jnp
from jax import lax
from jax.experimental import pallas as pl
from jax.experimental.pallas import tpu as pltpu
from jax.experimental.pallas import tpu_sc as plsc

D = 32
B = 4096
N = 1000000
PACK = 4
WIDE = PACK * D
SUBL = 4096
SHS = 12
NC, NS, L = 2, 16, 16
NW = NC * NS
BPW = B // NW
GROUPS = BPW // L

TRC = PACK * SUBL
TRG = -(-N // TRC)
NWR = TRG * SUBL

_sc_mesh = plsc.VectorSubcoreMesh(core_axis_name="c", subcore_axis_name="s")


def _tr_body(in_ref, o_ref):
    for q in range(PACK):
        o_ref[:, q * D:(q + 1) * D] = in_ref[:, q * SUBL:(q + 1) * SUBL].T


@jax.jit
def _to_wide(embT):
    return pl.pallas_call(
        _tr_body,
        grid=(TRG,),
        in_specs=[pl.BlockSpec((D, TRC), lambda i: (0, i))],
        out_specs=pl.BlockSpec((SUBL, WIDE), lambda i: (i, 0)),
        out_shape=jax.ShapeDtypeStruct((NWR, WIDE), jnp.float32),
    )(embT)


@functools.partial(
    pl.kernel,
    out_type=(
        jax.ShapeDtypeStruct((B, WIDE), jnp.float32),
        jax.ShapeDtypeStruct((B, WIDE), jnp.float32),
        jax.ShapeDtypeStruct((B,), jnp.float32),
    ),
    mesh=_sc_mesh,
    scratch_types=[
        pltpu.VMEM((BPW,), jnp.int32),
        pltpu.VMEM((BPW,), jnp.int32),
        pltpu.VMEM((BPW,), jnp.int32),
        pltpu.VMEM((BPW,), jnp.int32),
        pltpu.VMEM((BPW, WIDE), jnp.float32),
        pltpu.VMEM((BPW, WIDE), jnp.float32),
        pltpu.VMEM((BPW,), jnp.float32),
        pltpu.VMEM((BPW,), jnp.float32),
        pltpu.SemaphoreType.DMA,
        pltpu.SemaphoreType.DMA,
        pltpu.SemaphoreType.DMA,
        pltpu.SemaphoreType.DMA,
    ],
)
def _sc_gather(uemb2, iemb2, uids, iids, ubias, ibias,
               ue_out, ie_out, brow_out,
               uid_v, iid_v, utid_v, itid_v, uwide, iwide, ubv, ibv,
               sem_u, sem_i, sem_ub, sem_ib):
    wid = lax.axis_index("s") * NC + lax.axis_index("c")
    base = wid * BPW
    pltpu.sync_copy(uids.at[pl.ds(base, BPW)], uid_v)
    pltpu.sync_copy(iids.at[pl.ds(base, BPW)], iid_v)
    for g in range(GROUPS):
        s = pl.ds(g * L, L)
        u = uid_v[s]
        i = iid_v[s]
        utid_v[s] = lax.shift_left(lax.shift_right_logical(u, SHS + 2), SHS) \
            + (u & (SUBL - 1))
        itid_v[s] = lax.shift_left(lax.shift_right_logical(i, SHS + 2), SHS) \
            + (i & (SUBL - 1))
    cu = pltpu.async_copy(uemb2.at[utid_v], uwide, sem_u)
    ci = pltpu.async_copy(iemb2.at[itid_v], iwide, sem_i)
    cub = pltpu.async_copy(ubias.at[uid_v], ubv, sem_ub)
    cib = pltpu.async_copy(ibias.at[iid_v], ibv, sem_ib)
    cu.wait()
    pltpu.sync_copy(uwide, ue_out.at[pl.ds(base, BPW)])
    ci.wait()
    pltpu.sync_copy(iwide, ie_out.at[pl.ds(base, BPW)])
    cub.wait()
    cib.wait()
    for g in range(GROUPS):
        s = pl.ds(g * L, L)
        ubv[s] = ubv[s] + ibv[s]
    pltpu.sync_copy(ubv, brow_out.at[pl.ds(base, BPW)])


def _dot_body(uw_ref, iw_ref, uq_ref, iq_ref, o_ref):
    uq = lax.shift_right_logical(uq_ref[...], SHS) & 3
    iq = lax.shift_right_logical(iq_ref[...], SHS) & 3
    ue = jnp.zeros((B, D), jnp.float32)
    ie = jnp.zeros((B, D), jnp.float32)
    for q in range(PACK):
        sel = pl.ds(q * D, D)
        ue = ue + jnp.where(uq == q, uw_ref[:, sel], 0.0)
        ie = ie + jnp.where(iq == q, iw_ref[:, sel], 0.0)
    o_ref[...] = jnp.sum(ue * ie, axis=1, keepdims=True)


def _bcast_body(dot_ref, brow_ref, out_ref):
    out_ref[...] = brow_ref[...] + dot_ref[...]


TILE_I = 512


@jax.jit
def _tc_stage(uw, iw, uids, iids, brow):
    dot_col = pl.pallas_call(
        _dot_body,
        out_shape=jax.ShapeDtypeStruct((B, 1), jnp.float32),
    )(uw, iw, uids.reshape(B, 1), iids.reshape(B, 1))
    return pl.pallas_call(
        _bcast_body,
        grid=(B // TILE_I,),
        in_specs=[
            pl.BlockSpec((1, B), lambda i: (0, 0)),
            pl.BlockSpec((TILE_I, 1), lambda i: (i, 0)),
        ],
        out_specs=pl.BlockSpec((TILE_I, B), lambda i: (i, 0)),
        out_shape=jax.ShapeDtypeStruct((B, B), jnp.float32),
    )(dot_col.reshape(1, B), brow.reshape(B, 1))


def kernel(user_ids, item_ids, user_emb, item_emb, user_bias, item_bias):
    uids = user_ids.astype(jnp.int32)
    iids = item_ids.astype(jnp.int32)
    uw2 = _to_wide(user_emb.T)
    iw2 = _to_wide(item_emb.T)
    uw, iw, brow = _sc_gather(
        uw2, iw2, uids, iids,
        user_bias.reshape(-1), item_bias.reshape(-1))
    return _tc_stage(uw, iw, uids, iids, brow)
